# Initial kernel scaffold; baseline (speedup 1.0000x reference)
import jax
import jax.numpy as jnp
from jax.experimental import pallas as pl


def kernel(x, input_edge_attr, d2an, W_node, W1, W2, input_edge_index, firstLayer):
    raise NotImplementedError("write your pallas kernel here")


# speedup vs baseline: 5.8308x; 5.8308x over previous
import functools

import jax
import jax.numpy as jnp
from jax import lax
from jax.experimental import pallas as pl
from jax.experimental.pallas import tpu as pltpu
from jax.experimental.pallas import tpu_sc as plsc

NC = 2
NS = 16
L = 16


def _sc_mesh():
  return plsc.VectorSubcoreMesh(core_axis_name="c", subcore_axis_name="s")


def _make_deg_kernel(E, Np, CH):
  per_tile = E // (NC * NS)
  n_chunks = per_tile // CH
  T = Np // NS

  def body(col_hbm, out_hbm, colv, ones_v, zb, acc_sh):
    cid = lax.axis_index("c")
    sid = lax.axis_index("s")
    w = cid * NS + sid

    for i in range(CH // L):
      ones_v[pl.ds(i * L, L)] = jnp.ones((L,), jnp.float32)
    for i in range(T // L):
      zb[pl.ds(i * L, L)] = jnp.zeros((L,), jnp.float32)
    pltpu.sync_copy(zb, acc_sh.at[pl.ds(sid * T, T)])
    plsc.subcore_barrier()

    def chunk(i, carry):
      base = w * per_tile + i * CH
      pltpu.sync_copy(col_hbm.at[pl.ds(base, CH)], colv)
      pltpu.sync_copy(ones_v, acc_sh.at[colv], add=True)
      return carry

    lax.fori_loop(0, n_chunks, chunk, 0)
    plsc.subcore_barrier()
    pltpu.sync_copy(acc_sh.at[pl.ds(sid * T, T)],
                    out_hbm.at[cid, pl.ds(sid * T, T)])

  return pl.kernel(
      body,
      out_type=jax.ShapeDtypeStruct((NC, Np), jnp.float32),
      mesh=_sc_mesh(),
      scratch_types=[
          pltpu.VMEM((CH,), jnp.int32),
          pltpu.VMEM((CH,), jnp.float32),
          pltpu.VMEM((T,), jnp.float32),
          pltpu.VMEM_SHARED((Np,), jnp.float32),
      ],
  )


def _make_agg_kernel(E, Np, D, CH):
  per_tile = E // NS
  n_chunks = per_tile // CH
  T = Np // NS

  def body(y_hbm, xn_hbm, row_hbm, col_hbm, a2_hbm, out_hbm,
           rowv, colv, a2v, rows, acc_sh, sem):
    cid = lax.axis_index("c")
    sid = lax.axis_index("s")

    def zrow(e, carry):
      for f in range(D // L):
        rows[e, pl.ds(f * L, L)] = jnp.zeros((L,), jnp.float32)
      return carry

    lax.fori_loop(0, CH, zrow, 0)
    for k in range(T // CH):
      pltpu.sync_copy(rows, acc_sh.at[pl.ds(sid * T + k * CH, CH)])
    plsc.subcore_barrier()

    def chunk(i, carry):
      base = sid * per_tile + i * CH
      pltpu.sync_copy(row_hbm.at[pl.ds(base, CH)], rowv)
      pltpu.sync_copy(col_hbm.at[pl.ds(base, CH)], colv)

      @pl.when(cid == 0)
      def _():
        pltpu.async_copy(y_hbm.at[rowv], rows, sem).wait()

      @pl.when(cid == 1)
      def _():
        pltpu.async_copy(xn_hbm.at[rowv], rows, sem).wait()
        pltpu.sync_copy(a2_hbm.at[pl.ds(base, CH)], a2v)

        def scale(e, c2):
          s = a2v[e, :]
          for f in range(D // L):
            rows[e, pl.ds(f * L, L)] = rows[e, pl.ds(f * L, L)] * s
          return c2

        lax.fori_loop(0, CH, scale, 0)

      pltpu.sync_copy(rows, acc_sh.at[colv], add=True)
      return carry

    lax.fori_loop(0, n_chunks, chunk, 0)
    plsc.subcore_barrier()
    pltpu.sync_copy(acc_sh.at[pl.ds(sid * T, T)],
                    out_hbm.at[cid, pl.ds(sid * T, T)])

  return pl.kernel(
      body,
      out_type=jax.ShapeDtypeStruct((NC, Np, D), jnp.float32),
      mesh=_sc_mesh(),
      scratch_types=[
          pltpu.VMEM((CH,), jnp.int32),
          pltpu.VMEM((CH,), jnp.int32),
          pltpu.VMEM((CH, L), jnp.float32),
          pltpu.VMEM((CH, D), jnp.float32),
          pltpu.VMEM_SHARED((Np, D), jnp.float32),
          pltpu.SemaphoreType.DMA,
      ],
  )


def _prep_body(fl_ref, x_ref, d2_ref, wnx_ref, wnd_ref, degp_ref,
               xn_ref, y_ref):
  x = x_ref[...]
  xn = lax.dot_general(x, wnx_ref[...], (((1,), (1,)), ((), ())),
                       preferred_element_type=jnp.float32)
  xn += lax.dot_general(d2_ref[...], wnd_ref[...], (((1,), (1,)), ((), ())),
                        preferred_element_type=jnp.float32)
  xn = jnp.where(fl_ref[0, 0] != 0, xn, x)
  deg = jnp.sum(degp_ref[0], axis=1, keepdims=True) + 1.0
  dis = lax.rsqrt(deg)
  xn_ref[...] = xn
  y_ref[...] = xn * dis


def _make_prep_kernel(N, D, PE, BM):
  grid = (N // BM,)
  return pl.pallas_call(
      _prep_body,
      grid=grid,
      in_specs=[
          pl.BlockSpec(memory_space=pltpu.SMEM),
          pl.BlockSpec((BM, D), lambda i: (i, 0)),
          pl.BlockSpec((BM, PE), lambda i: (i, 0)),
          pl.BlockSpec((D, D), lambda i: (0, 0)),
          pl.BlockSpec((D, PE), lambda i: (0, 0)),
          pl.BlockSpec((1, BM, NC), lambda i: (i, 0, 0)),
      ],
      out_specs=[
          pl.BlockSpec((BM, D), lambda i: (i, 0)),
          pl.BlockSpec((BM, D), lambda i: (i, 0)),
      ],
      out_shape=[
          jax.ShapeDtypeStruct((N, D), jnp.float32),
          jax.ShapeDtypeStruct((N, D), jnp.float32),
      ],
  )


def _a2_body(attr_ref, a2_ref):
  a = attr_ref[...]
  a2_ref[...] = jnp.where(a > 0.0, jnp.minimum(lax.rsqrt(a), 1.0), 0.0)


def _make_a2_kernel(rows, cols):
  return pl.pallas_call(
      _a2_body,
      out_shape=jax.ShapeDtypeStruct((rows, cols), jnp.float32),
  )


def _final_body(accA_ref, accB_ref, xn_ref, degp_ref, w1_ref, w2_ref,
                out_ref):
  deg = jnp.sum(degp_ref[0], axis=1, keepdims=True) + 1.0
  dis = lax.rsqrt(deg)
  xn = xn_ref[...]
  t1 = accA_ref[...] * dis + xn * (1.0 / deg)
  t2 = accB_ref[...] + xn
  out = lax.dot_general(t1, w1_ref[...], (((1,), (1,)), ((), ())),
                        preferred_element_type=jnp.float32)
  out += lax.dot_general(t2, w2_ref[...], (((1,), (1,)), ((), ())),
                         preferred_element_type=jnp.float32)
  out_ref[...] = out


def _make_final_kernel(N, D, BM):
  grid = (N // BM,)
  return pl.pallas_call(
      _final_body,
      grid=grid,
      in_specs=[
          pl.BlockSpec((BM, D), lambda i: (i, 0)),
          pl.BlockSpec((BM, D), lambda i: (i, 0)),
          pl.BlockSpec((BM, D), lambda i: (i, 0)),
          pl.BlockSpec((1, BM, NC), lambda i: (i, 0, 0)),
          pl.BlockSpec((D, D), lambda i: (0, 0)),
          pl.BlockSpec((D, D), lambda i: (0, 0)),
      ],
      out_specs=pl.BlockSpec((BM, D), lambda i: (i, 0)),
      out_shape=jax.ShapeDtypeStruct((N, D), jnp.float32),
  )


@jax.jit
def kernel(x, input_edge_attr, d2an, W_node, W1, W2, input_edge_index,
           firstLayer):
  N, D = x.shape
  PE = d2an.shape[1]
  E = input_edge_index.shape[1]
  Np = 10240
  CH = 80
  BM = 1000

  row = input_edge_index[0]
  col = input_edge_index[1]
  fl = jnp.asarray(firstLayer, jnp.int32).reshape(1, 1)

  deg_parts = _make_deg_kernel(E, Np, CH)(col)
  degp = deg_parts.T[:N].reshape(N // BM, BM, NC)

  xn, y = _make_prep_kernel(N, D, PE, BM)(
      fl, x, d2an, W_node[:, :D], W_node[:, D:], degp)
  a2 = _make_a2_kernel(E // 128, 128)(
      input_edge_attr.reshape(E // 128, 128)).reshape(E)
  a2x = jnp.broadcast_to(a2[:, None], (E, L))

  acc = _make_agg_kernel(E, Np, D, CH)(y, xn, row, col, a2x)

  return _make_final_kernel(N, D, BM)(
      acc[0, :N], acc[1, :N], xn, degp, W1, W2)
